# baseline (speedup 1.0000x reference)
import jax
import jax.numpy as jnp
import numpy as np
from jax import lax
from jax.experimental import pallas as pl
from jax.experimental.pallas import tpu as pltpu
from jax.experimental.pallas import tpu_sc as plsc

_NUM_FIELDS = 26
_FIELD_DIM = 38462
_EMBED_DIM = 16
_BATCH = 16384
_TOTAL = _NUM_FIELDS * _FIELD_DIM

_NC, _NS = 2, 16
_NW = _NC * _NS
_B_PER_W = _BATCH // _NW
_N_PER_W = _B_PER_W * _NUM_FIELDS
_CHUNK = 128
_G_PER_W = _N_PER_W // _CHUNK
_K = 8
_M_PER_W = _G_PER_W // _K
_ROWS_PER_M = _K * _CHUNK
_C_PER_F = _B_PER_W // _CHUNK

_OFF_LOCAL = np.tile(
    np.arange(_NUM_FIELDS, dtype=np.int32) * _FIELD_DIM, _B_PER_W
).reshape(_G_PER_W, _CHUNK)[:13]


def _body(x_hbm, xt_hbm, off_hbm, bias_hbm, emb_hbm, fc_hbm, oemb_hbm,
          olr_hbm, idx_v, off_v, xt_v, obuf, fcbuf, lrbuf, bias_v, gsem,
          fsem):
    wid = lax.axis_index("s") * _NC + lax.axis_index("c")
    rbase = wid * _N_PER_W
    bbase = wid * _B_PER_W

    pltpu.sync_copy(x_hbm.at[pl.ds(wid * _G_PER_W, _G_PER_W)], idx_v)
    pltpu.sync_copy(xt_hbm.at[:, pl.ds(bbase, _B_PER_W)], xt_v)
    pltpu.sync_copy(off_hbm, off_v)
    pltpu.sync_copy(bias_hbm, bias_v)

    def _add_off(o, carry):
        for i in range(13):
            g = o * 13 + i
            for j in range(_CHUNK // 16):
                sl = pl.ds(j * 16, 16)
                idx_v[g, sl] = idx_v[g, sl] + off_v[i, sl]
        return carry

    lax.fori_loop(0, _G_PER_W // 13, _add_off, 0)

    def _add_off_t(t, carry):
        sl = pl.ds(t * 16, 16)
        for f in range(_NUM_FIELDS):
            xt_v[f, sl] = xt_v[f, sl] + (f * _FIELD_DIM)
        return carry

    lax.fori_loop(0, _B_PER_W // 16, _add_off_t, 0)

    def _macro(m, carry):
        descs = []
        for j in range(_K):
            g = m * _K + j
            descs.append(pltpu.async_copy(
                emb_hbm.at[idx_v.at[g]], obuf.at[pl.ds(j * _CHUNK, _CHUNK)],
                gsem))
        for d in descs:
            d.wait()
        pltpu.sync_copy(
            obuf, oemb_hbm.at[pl.ds(rbase + m * _ROWS_PER_M, _ROWS_PER_M)])
        return carry

    lax.fori_loop(0, _M_PER_W, _macro, 0)

    def _fc_gather(c, carry):
        descs = []
        for f in range(_NUM_FIELDS):
            descs.append(pltpu.async_copy(
                fc_hbm.at[xt_v.at[f, pl.ds(c * _CHUNK, _CHUNK)]],
                fcbuf.at[pl.ds(f * _B_PER_W + c * _CHUNK, _CHUNK)],
                fsem))
        for d in descs:
            d.wait()
        return carry

    lax.fori_loop(0, _C_PER_F, _fc_gather, 0)

    def _reduce(t, carry):
        acc = bias_v[...]
        for f in range(_NUM_FIELDS):
            acc = acc + fcbuf[pl.ds(f * _B_PER_W + t * 16, 16)]
        lrbuf[pl.ds(t * 16, 16)] = acc
        return carry

    lax.fori_loop(0, _B_PER_W // 16, _reduce, 0)
    pltpu.sync_copy(lrbuf, olr_hbm.at[pl.ds(bbase, _B_PER_W)])


def kernel(x, emb_table, fc_table, bias):
    x2 = x.reshape(_NW * _G_PER_W, _CHUNK)
    xt = x.T
    off2 = jnp.asarray(_OFF_LOCAL)
    bias16 = jnp.broadcast_to(bias, (16,))
    fc_flat = fc_table.reshape(_TOTAL)

    mesh = plsc.VectorSubcoreMesh(core_axis_name="c", subcore_axis_name="s",
                                  num_cores=_NC, num_subcores=_NS)
    run = pl.kernel(
        _body,
        out_type=[
            jax.ShapeDtypeStruct((_BATCH * _NUM_FIELDS, _EMBED_DIM),
                                 jnp.float32),
            jax.ShapeDtypeStruct((_BATCH,), jnp.float32),
        ],
        mesh=mesh,
        compiler_params=pltpu.CompilerParams(use_tc_tiling_on_sc=False),
        scratch_types=[
            pltpu.VMEM((_G_PER_W, _CHUNK), jnp.int32),
            pltpu.VMEM((13, _CHUNK), jnp.int32),
            pltpu.VMEM((_NUM_FIELDS, _B_PER_W), jnp.int32),
            pltpu.VMEM((_ROWS_PER_M, _EMBED_DIM), jnp.float32),
            pltpu.VMEM((_N_PER_W,), jnp.float32),
            pltpu.VMEM((_B_PER_W,), jnp.float32),
            pltpu.VMEM((16,), jnp.float32),
            pltpu.SemaphoreType.DMA,
            pltpu.SemaphoreType.DMA,
        ],
    )
    oemb, olr = run(x2, xt, off2, bias16, emb_table, fc_flat)
    return oemb.reshape(_BATCH, _NUM_FIELDS, _EMBED_DIM), olr.reshape(_BATCH, 1)


# speedup vs baseline: 1.4240x; 1.4240x over previous
import jax
import jax.numpy as jnp
from jax import lax
from jax.experimental import pallas as pl
from jax.experimental.pallas import tpu as pltpu
from jax.experimental.pallas import tpu_sc as plsc

_NUM_FIELDS = 26
_FIELD_DIM = 38462
_EMBED_DIM = 16
_BATCH = 16384
_TOTAL = _NUM_FIELDS * _FIELD_DIM

_NC, _NS = 2, 16
_NW = _NC * _NS
_B_PER_W = _BATCH // _NW
_CHUNK = 128
_C_PER_F = _B_PER_W // _CHUNK
_ROWS_W = _NUM_FIELDS * _C_PER_F


def _body(xt_hbm, bias_hbm, emb_hbm, fc_hbm, oemb_hbm, olr_hbm,
          xt_v, fbuf0, fbuf1, fcbuf, lrbuf, bias_v, gsem, fsem):
    wid = lax.axis_index("s") * _NC + lax.axis_index("c")
    bbase = wid * _B_PER_W
    crow = wid * _C_PER_F

    for f in range(_NUM_FIELDS):
        pltpu.sync_copy(
            xt_hbm.at[pl.ds(f * (_BATCH // _CHUNK) + crow, _C_PER_F)],
            xt_v.at[pl.ds(f * _C_PER_F, _C_PER_F)])
    pltpu.sync_copy(bias_hbm, bias_v)

    def _add_off(c, carry):
        for f in range(_NUM_FIELDS):
            for j in range(_CHUNK // 16):
                sl = pl.ds(j * 16, 16)
                r = f * _C_PER_F + c
                xt_v[r, sl] = xt_v[r, sl] + (f * _FIELD_DIM)
        return carry

    lax.fori_loop(0, _C_PER_F, _add_off, 0)

    bufs = (fbuf0, fbuf1)

    def _issue(f, buf):
        descs = []
        for c in range(_C_PER_F):
            descs.append(pltpu.async_copy(
                emb_hbm.at[xt_v.at[f * _C_PER_F + c]],
                buf.at[pl.ds(c * _CHUNK, _CHUNK)], gsem))
        return descs

    d_cur = _issue(0, fbuf0)
    for f in range(_NUM_FIELDS):
        d_next = _issue(f + 1, bufs[(f + 1) % 2]) if f + 1 < _NUM_FIELDS else []
        for d in d_cur:
            d.wait()
        pltpu.sync_copy(bufs[f % 2],
                        oemb_hbm.at[f, pl.ds(bbase, _B_PER_W)])
        d_cur = d_next

    def _fc_gather(c, carry):
        descs = []
        for f in range(_NUM_FIELDS):
            descs.append(pltpu.async_copy(
                fc_hbm.at[xt_v.at[f * _C_PER_F + c]],
                fcbuf.at[pl.ds(f * _B_PER_W + c * _CHUNK, _CHUNK)], fsem))
        for d in descs:
            d.wait()
        return carry

    lax.fori_loop(0, _C_PER_F, _fc_gather, 0)

    def _reduce(t, carry):
        acc = bias_v[...]
        for f in range(_NUM_FIELDS):
            acc = acc + fcbuf[pl.ds(f * _B_PER_W + t * 16, 16)]
        lrbuf[pl.ds(t * 16, 16)] = acc
        return carry

    lax.fori_loop(0, _B_PER_W // 16, _reduce, 0)
    pltpu.sync_copy(lrbuf, olr_hbm.at[pl.ds(bbase, _B_PER_W)])


def kernel(x, emb_table, fc_table, bias):
    xt2 = x.T.reshape(_BATCH * _NUM_FIELDS // _CHUNK, _CHUNK)
    bias16 = jnp.broadcast_to(bias, (16,))
    fc_flat = fc_table.reshape(_TOTAL)

    mesh = plsc.VectorSubcoreMesh(core_axis_name="c", subcore_axis_name="s",
                                  num_cores=_NC, num_subcores=_NS)
    run = pl.kernel(
        _body,
        out_type=[
            jax.ShapeDtypeStruct((_NUM_FIELDS, _BATCH, _EMBED_DIM),
                                 jnp.float32),
            jax.ShapeDtypeStruct((_BATCH,), jnp.float32),
        ],
        mesh=mesh,
        compiler_params=pltpu.CompilerParams(use_tc_tiling_on_sc=False),
        scratch_types=[
            pltpu.VMEM((_ROWS_W, _CHUNK), jnp.int32),
            pltpu.VMEM((_B_PER_W, _EMBED_DIM), jnp.float32),
            pltpu.VMEM((_B_PER_W, _EMBED_DIM), jnp.float32),
            pltpu.VMEM((_NUM_FIELDS * _B_PER_W,), jnp.float32),
            pltpu.VMEM((_B_PER_W,), jnp.float32),
            pltpu.VMEM((16,), jnp.float32),
            pltpu.SemaphoreType.DMA,
            pltpu.SemaphoreType.DMA,
        ],
    )
    oemb, olr = run(xt2, bias16, emb_table, fc_flat)
    return jnp.transpose(oemb, (1, 0, 2)), olr.reshape(_BATCH, 1)


# speedup vs baseline: 1.4723x; 1.0339x over previous
import jax
import jax.numpy as jnp
from jax import lax
from jax.experimental import pallas as pl
from jax.experimental.pallas import tpu as pltpu
from jax.experimental.pallas import tpu_sc as plsc

_NUM_FIELDS = 26
_FIELD_DIM = 38462
_EMBED_DIM = 16
_BATCH = 16384
_TOTAL = _NUM_FIELDS * _FIELD_DIM

_NC, _NS = 2, 16
_NW = _NC * _NS
_B_PER_W = _BATCH // _NW
_CHUNK = 128
_C_PER_F = _B_PER_W // _CHUNK
_ROWS_W = _NUM_FIELDS * _C_PER_F


def _body(xt_hbm, bias_hbm, emb_hbm, fc_hbm, oemb_hbm, olr_hbm,
          xt_v, fbuf0, fbuf1, fcbuf, lrbuf, bias_v, gsem, fsem, wsem):
    wid = lax.axis_index("s") * _NC + lax.axis_index("c")
    bbase = wid * _B_PER_W
    crow = wid * _C_PER_F

    st_descs = [
        pltpu.async_copy(
            xt_hbm.at[pl.ds(f * (_BATCH // _CHUNK) + crow, _C_PER_F)],
            xt_v.at[pl.ds(f * _C_PER_F, _C_PER_F)], wsem)
        for f in range(_NUM_FIELDS)
    ]
    pltpu.sync_copy(bias_hbm, bias_v)
    for d in st_descs:
        d.wait()

    def _add_off(c, carry):
        for f in range(_NUM_FIELDS):
            for j in range(_CHUNK // 16):
                sl = pl.ds(j * 16, 16)
                r = f * _C_PER_F + c
                xt_v[r, sl] = xt_v[r, sl] + (f * _FIELD_DIM)
        return carry

    lax.fori_loop(0, _C_PER_F, _add_off, 0)

    bufs = (fbuf0, fbuf1)

    def _issue(f, buf):
        descs = []
        for c in range(_C_PER_F):
            descs.append(pltpu.async_copy(
                emb_hbm.at[xt_v.at[f * _C_PER_F + c]],
                buf.at[pl.ds(c * _CHUNK, _CHUNK)], gsem))
        return descs

    def _issue_fc(f):
        descs = []
        for c in range(_C_PER_F):
            descs.append(pltpu.async_copy(
                fc_hbm.at[xt_v.at[f * _C_PER_F + c]],
                fcbuf.at[pl.ds(f * _B_PER_W + c * _CHUNK, _CHUNK)], fsem))
        return descs

    fc_descs = []
    wb_descs = [None, None]
    d_cur = _issue(0, fbuf0)
    fc_descs += _issue_fc(0)
    for f in range(_NUM_FIELDS):
        if f + 1 < _NUM_FIELDS:
            d_next = _issue(f + 1, bufs[(f + 1) % 2])
            fc_descs += _issue_fc(f + 1)
        else:
            d_next = []
        for d in d_cur:
            d.wait()
        if wb_descs[f % 2] is not None:
            wb_descs[f % 2].wait()
        wb_descs[f % 2] = pltpu.async_copy(
            bufs[f % 2], oemb_hbm.at[f, pl.ds(bbase, _B_PER_W)], wsem)
        d_cur = d_next
    for d in wb_descs:
        if d is not None:
            d.wait()
    for d in fc_descs:
        d.wait()

    def _reduce(t, carry):
        acc = bias_v[...]
        for f in range(_NUM_FIELDS):
            acc = acc + fcbuf[pl.ds(f * _B_PER_W + t * 16, 16)]
        lrbuf[pl.ds(t * 16, 16)] = acc
        return carry

    lax.fori_loop(0, _B_PER_W // 16, _reduce, 0)
    pltpu.sync_copy(lrbuf, olr_hbm.at[pl.ds(bbase, _B_PER_W)])


def kernel(x, emb_table, fc_table, bias):
    xt2 = x.T.reshape(_BATCH * _NUM_FIELDS // _CHUNK, _CHUNK)
    bias16 = jnp.broadcast_to(bias, (16,))
    fc_flat = fc_table.reshape(_TOTAL)

    mesh = plsc.VectorSubcoreMesh(core_axis_name="c", subcore_axis_name="s",
                                  num_cores=_NC, num_subcores=_NS)
    run = pl.kernel(
        _body,
        out_type=[
            jax.ShapeDtypeStruct((_NUM_FIELDS, _BATCH, _EMBED_DIM),
                                 jnp.float32),
            jax.ShapeDtypeStruct((_BATCH,), jnp.float32),
        ],
        mesh=mesh,
        compiler_params=pltpu.CompilerParams(use_tc_tiling_on_sc=False),
        scratch_types=[
            pltpu.VMEM((_ROWS_W, _CHUNK), jnp.int32),
            pltpu.VMEM((_B_PER_W, _EMBED_DIM), jnp.float32),
            pltpu.VMEM((_B_PER_W, _EMBED_DIM), jnp.float32),
            pltpu.VMEM((_NUM_FIELDS * _B_PER_W,), jnp.float32),
            pltpu.VMEM((_B_PER_W,), jnp.float32),
            pltpu.VMEM((16,), jnp.float32),
            pltpu.SemaphoreType.DMA,
            pltpu.SemaphoreType.DMA,
            pltpu.SemaphoreType.DMA,
        ],
    )
    oemb, olr = run(xt2, bias16, emb_table, fc_flat)
    return jnp.transpose(oemb, (1, 0, 2)), olr.reshape(_BATCH, 1)


# speedup vs baseline: 1.4787x; 1.0044x over previous
import jax
import jax.numpy as jnp
from jax import lax
from jax.experimental import pallas as pl
from jax.experimental.pallas import tpu as pltpu
from jax.experimental.pallas import tpu_sc as plsc

_NUM_FIELDS = 26
_FIELD_DIM = 38462
_EMBED_DIM = 16
_BATCH = 16384
_TOTAL = _NUM_FIELDS * _FIELD_DIM

_NC, _NS = 2, 16
_NW = _NC * _NS
_B_PER_W = _BATCH // _NW
_CHUNK = 128
_C_PER_F = _B_PER_W // _CHUNK
_ROWS_W = _NUM_FIELDS * _C_PER_F


def _body(xt_hbm, bias_hbm, emb_hbm, fc_hbm, oemb_hbm, olr_hbm,
          xt_v, fbuf0, fbuf1, fbuf2, fcbuf, lrbuf, bias_v, gsem, fsem, wsem):
    wid = lax.axis_index("s") * _NC + lax.axis_index("c")
    bbase = wid * _B_PER_W
    crow = wid * _C_PER_F

    st_descs = [
        pltpu.async_copy(
            xt_hbm.at[pl.ds(f * (_BATCH // _CHUNK) + crow, _C_PER_F)],
            xt_v.at[pl.ds(f * _C_PER_F, _C_PER_F)], wsem)
        for f in range(_NUM_FIELDS)
    ]
    pltpu.sync_copy(bias_hbm, bias_v)
    for d in st_descs:
        d.wait()

    def _add_off(c, carry):
        for f in range(_NUM_FIELDS):
            for j in range(_CHUNK // 16):
                sl = pl.ds(j * 16, 16)
                r = f * _C_PER_F + c
                xt_v[r, sl] = xt_v[r, sl] + (f * _FIELD_DIM)
        return carry

    lax.fori_loop(0, _C_PER_F, _add_off, 0)

    bufs = (fbuf0, fbuf1, fbuf2)
    _NBUF = len(bufs)

    def _issue(f, buf):
        descs = []
        for c in range(_C_PER_F):
            descs.append(pltpu.async_copy(
                emb_hbm.at[xt_v.at[f * _C_PER_F + c]],
                buf.at[pl.ds(c * _CHUNK, _CHUNK)], gsem))
        return descs

    def _issue_fc(f):
        descs = []
        for c in range(_C_PER_F):
            descs.append(pltpu.async_copy(
                fc_hbm.at[xt_v.at[f * _C_PER_F + c]],
                fcbuf.at[pl.ds(f * _B_PER_W + c * _CHUNK, _CHUNK)], fsem))
        return descs

    fc_descs = []
    wb_descs = [None] * _NBUF
    g_descs = [None] * _NBUF
    for f in range(_NBUF - 1):
        g_descs[f % _NBUF] = _issue(f, bufs[f % _NBUF])
        fc_descs += _issue_fc(f)
    for f in range(_NUM_FIELDS):
        nf = f + _NBUF - 1
        if nf < _NUM_FIELDS:
            if wb_descs[nf % _NBUF] is not None:
                wb_descs[nf % _NBUF].wait()
                wb_descs[nf % _NBUF] = None
            g_descs[nf % _NBUF] = _issue(nf, bufs[nf % _NBUF])
            fc_descs += _issue_fc(nf)
        for d in g_descs[f % _NBUF]:
            d.wait()
        wb_descs[f % _NBUF] = pltpu.async_copy(
            bufs[f % _NBUF], oemb_hbm.at[f, pl.ds(bbase, _B_PER_W)], wsem)
    for d in wb_descs:
        if d is not None:
            d.wait()
    for d in fc_descs:
        d.wait()

    def _reduce(t, carry):
        acc = bias_v[...]
        for f in range(_NUM_FIELDS):
            acc = acc + fcbuf[pl.ds(f * _B_PER_W + t * 16, 16)]
        lrbuf[pl.ds(t * 16, 16)] = acc
        return carry

    lax.fori_loop(0, _B_PER_W // 16, _reduce, 0)
    pltpu.sync_copy(lrbuf, olr_hbm.at[pl.ds(bbase, _B_PER_W)])


def kernel(x, emb_table, fc_table, bias):
    xt2 = x.T.reshape(_BATCH * _NUM_FIELDS // _CHUNK, _CHUNK)
    bias16 = jnp.broadcast_to(bias, (16,))
    fc_flat = fc_table.reshape(_TOTAL)

    mesh = plsc.VectorSubcoreMesh(core_axis_name="c", subcore_axis_name="s",
                                  num_cores=_NC, num_subcores=_NS)
    run = pl.kernel(
        _body,
        out_type=[
            jax.ShapeDtypeStruct((_NUM_FIELDS, _BATCH, _EMBED_DIM),
                                 jnp.float32),
            jax.ShapeDtypeStruct((_BATCH,), jnp.float32),
        ],
        mesh=mesh,
        compiler_params=pltpu.CompilerParams(use_tc_tiling_on_sc=False),
        scratch_types=[
            pltpu.VMEM((_ROWS_W, _CHUNK), jnp.int32),
            pltpu.VMEM((_B_PER_W, _EMBED_DIM), jnp.float32),
            pltpu.VMEM((_B_PER_W, _EMBED_DIM), jnp.float32),
            pltpu.VMEM((_B_PER_W, _EMBED_DIM), jnp.float32),
            pltpu.VMEM((_NUM_FIELDS * _B_PER_W,), jnp.float32),
            pltpu.VMEM((_B_PER_W,), jnp.float32),
            pltpu.VMEM((16,), jnp.float32),
            pltpu.SemaphoreType.DMA,
            pltpu.SemaphoreType.DMA,
            pltpu.SemaphoreType.DMA,
        ],
    )
    oemb, olr = run(xt2, bias16, emb_table, fc_flat)
    return jnp.transpose(oemb, (1, 0, 2)), olr.reshape(_BATCH, 1)
